# baseline (speedup 1.0000x reference)
import functools

import jax
import jax.numpy as jnp
from jax import lax
from jax.experimental import pallas as pl
from jax.experimental.pallas import tpu as pltpu
from jax.experimental.pallas import tpu_sc as plsc

NC = 2
NS = 16
NW = NC * NS
LANES = 16


@functools.lru_cache(maxsize=None)
def _build(B, M, D):
    bw = B // NW
    assert bw * NW == B and bw == 128
    assert D % 8 == 0
    ng = bw // LANES
    pc = 4
    chunk = pc * bw
    nchunk = M // pc
    assert nchunk * pc == M
    npiece = 4
    prows = bw // npiece
    pg = prows // LANES

    mesh = plsc.VectorSubcoreMesh(core_axis_name="c", subcore_axis_name="s")

    @functools.partial(
        pl.kernel,
        out_type=jax.ShapeDtypeStruct((M, D // 8, B // 128, 8, 128),
                                      jnp.float32),
        mesh=mesh,
        compiler_params=pltpu.CompilerParams(use_tc_tiling_on_sc=False, needs_layout_passes=False),
        scratch_types=[
            pltpu.VMEM((prows, M), jnp.int32),
            pltpu.VMEM((M * bw,), jnp.int32),
            pltpu.VMEM((chunk, D), jnp.float32),
            pltpu.VMEM((chunk, D), jnp.float32),
            pltpu.VMEM((pc, D // 8, 8, bw), jnp.float32),
            pltpu.VMEM((pc, D // 8, 8, bw), jnp.float32),
            pltpu.VMEM((M, D), jnp.float32),
            pltpu.SemaphoreType.DMA,
            pltpu.SemaphoreType.DMA,
            pltpu.SemaphoreType.DMA,
            pltpu.SemaphoreType.DMA,
        ],
    )
    def k(x_hbm, tab_hbm, pos_hbm, out_hbm, xv, xt, rows0, rows1,
          tr0, tr1, pos_v, gsem0, gsem1, ssem0, ssem1):
        cid = lax.axis_index("c")
        sid = lax.axis_index("s")
        wid = sid * NC + cid
        b0 = wid * bw

        rows = (rows0, rows1)
        tr = (tr0, tr1)
        gsems = (gsem0, gsem1)
        ssems = (ssem0, ssem1)

        iota = jax.lax.iota(jnp.int32, LANES)
        pltpu.sync_copy(pos_hbm, pos_v)

        for q in range(npiece):
            pltpu.sync_copy(x_hbm.at[pl.ds(b0 + q * prows, prows)], xv)

            def xbody(p, carry, q=q):
                for gg in range(pg):
                    v = plsc.load_gather(xv, [gg * LANES + iota,
                                              jnp.broadcast_to(p, (LANES,))])
                    xt[pl.ds(p * bw + (q * pg + gg) * LANES, LANES)] = v
                return carry

            lax.fori_loop(0, M, xbody, None)

        def start_gather(bb, c):
            return pltpu.async_copy(
                tab_hbm.at[xt.at[pl.ds(c * chunk, chunk)]], rows[bb],
                gsems[bb])

        def wait_gather(bb, c):
            pltpu.make_async_copy(
                tab_hbm.at[xt.at[pl.ds(c * chunk, chunk)]], rows[bb],
                gsems[bb]).wait()

        def start_store(bb, c):
            return pltpu.async_copy(
                tr[bb], out_hbm.at[pl.ds(c * pc, pc), :, wid], ssems[bb])

        def wait_store(bb, c):
            pltpu.make_async_copy(
                tr[bb], out_hbm.at[pl.ds(c * pc, pc), :, wid],
                ssems[bb]).wait()

        def transpose_add(bb, c):
            def body(i, carry):
                pl_ = i >> 5
                ch = i & 31
                pvec = plsc.load_gather(
                    pos_v, [jnp.broadcast_to(c * pc + pl_, (LANES,)),
                            jnp.broadcast_to(ch, (LANES,))])
                for g in range(ng):
                    t0 = pl_ * bw + g * LANES
                    v = plsc.load_gather(
                        rows[bb], [t0 + iota,
                                   jnp.broadcast_to(ch, (LANES,))])
                    tr[bb][pl_, ch >> 3, ch & 7,
                           pl.ds(g * LANES, LANES)] = v + pvec
                return carry

            lax.fori_loop(0, pc * D, body, None)

        start_gather(0, 0)
        start_gather(1, 1)
        wait_gather(0, 0)
        transpose_add(0, 0)
        start_store(0, 0)
        start_gather(0, 2)
        wait_gather(1, 1)
        transpose_add(1, 1)
        start_store(1, 1)

        def pair(cc, carry):
            c = cc * 2
            for b in range(2):
                wait_store(b, c + b - 2)
                start_gather(1 - b, c + b + 1)
                wait_gather(b, c + b)
                transpose_add(b, c + b)
                start_store(b, c + b)
            return carry

        lax.fori_loop(1, nchunk // 2 - 1, pair, None)
        c = nchunk - 2
        wait_store(0, c - 2)
        start_gather(1, c + 1)
        wait_gather(0, c)
        transpose_add(0, c)
        start_store(0, c)
        wait_store(1, c - 1)
        wait_gather(1, c + 1)
        transpose_add(1, c + 1)
        start_store(1, c + 1)
        wait_store(0, c)
        wait_store(1, c + 1)

    return k


def kernel(x, token_table, pos_table):
    B, M = x.shape
    D = token_table.shape[1]
    k = _build(B, M, D)
    out5d = k(x, token_table, pos_table)
    return jnp.transpose(out5d, (2, 4, 0, 1, 3)).reshape(B, M, D)


# speedup vs baseline: 1.6493x; 1.6493x over previous
import functools

import jax
import jax.numpy as jnp
from jax import lax
from jax.experimental import pallas as pl
from jax.experimental.pallas import tpu as pltpu
from jax.experimental.pallas import tpu_sc as plsc

NC = 2
NS = 16
NW = NC * NS
LANES = 16


@functools.lru_cache(maxsize=None)
def _build(B, M, D):
    bw = B // NW
    assert bw * NW == B and bw == 128
    assert D % 8 == 0
    ng = bw // LANES
    pc = 4
    chunk = pc * bw
    nchunk = M // pc
    assert nchunk * pc == M
    npiece = 4
    prows = bw // npiece
    pg = prows // LANES

    mesh = plsc.VectorSubcoreMesh(core_axis_name="c", subcore_axis_name="s")

    @functools.partial(
        pl.kernel,
        out_type=jax.ShapeDtypeStruct((M, D // 8, B // 128, 8, 128),
                                      jnp.float32),
        mesh=mesh,
        compiler_params=pltpu.CompilerParams(use_tc_tiling_on_sc=False, needs_layout_passes=False),
        scratch_types=[
            pltpu.VMEM((prows, M), jnp.int32),
            pltpu.VMEM((M * bw,), jnp.int32),
            pltpu.VMEM((chunk, D), jnp.float32),
            pltpu.VMEM((chunk, D), jnp.float32),
            pltpu.VMEM((chunk, D + 1), jnp.float32),
            pltpu.VMEM((pc, D // 8, 8, bw), jnp.float32),
            pltpu.VMEM((pc, D // 8, 8, bw), jnp.float32),
            pltpu.VMEM((M, D), jnp.float32),
            pltpu.SemaphoreType.DMA,
            pltpu.SemaphoreType.DMA,
            pltpu.SemaphoreType.DMA,
            pltpu.SemaphoreType.DMA,
        ],
    )
    def k(x_hbm, tab_hbm, pos_hbm, out_hbm, xv, xt, rows0, rows1, rpk,
          tr0, tr1, pos_v, gsem0, gsem1, ssem0, ssem1):
        cid = lax.axis_index("c")
        sid = lax.axis_index("s")
        wid = sid * NC + cid
        b0 = wid * bw

        rows = (rows0, rows1)
        tr = (tr0, tr1)
        gsems = (gsem0, gsem1)
        ssems = (ssem0, ssem1)

        iota = jax.lax.iota(jnp.int32, LANES)
        pltpu.sync_copy(pos_hbm, pos_v)

        for q in range(npiece):
            pltpu.sync_copy(x_hbm.at[pl.ds(b0 + q * prows, prows)], xv)

            def xbody(p, carry, q=q):
                for gg in range(pg):
                    v = plsc.load_gather(xv, [gg * LANES + iota,
                                              jnp.broadcast_to(p, (LANES,))])
                    xt[pl.ds(p * bw + (q * pg + gg) * LANES, LANES)] = v
                return carry

            lax.fori_loop(0, M, xbody, None)

        def start_gather(bb, c):
            return pltpu.async_copy(
                tab_hbm.at[xt.at[pl.ds(c * chunk, chunk)]], rows[bb],
                gsems[bb])

        def wait_gather(bb, c):
            pltpu.make_async_copy(
                tab_hbm.at[xt.at[pl.ds(c * chunk, chunk)]], rows[bb],
                gsems[bb]).wait()

        def start_store(bb, c):
            return pltpu.async_copy(
                tr[bb], out_hbm.at[pl.ds(c * pc, pc), :, wid], ssems[bb])

        def wait_store(bb, c):
            pltpu.make_async_copy(
                tr[bb], out_hbm.at[pl.ds(c * pc, pc), :, wid],
                ssems[bb]).wait()

        def transpose_add(bb, c):
            def rbody(j, carry):
                prow = c * pc + ((j * 8) >> 7)
                pv = [pos_v[prow, pl.ds(h * LANES, LANES)] for h in range(2)]
                vs = [rows[bb][j * 8 + u, pl.ds(h * LANES, LANES)] + pv[h]
                      for u in range(8) for h in range(2)]
                i2 = 0
                for u in range(8):
                    for h in range(2):
                        rpk[j * 8 + u, pl.ds(h * LANES, LANES)] = vs[i2]
                        i2 += 1
                return carry

            lax.fori_loop(0, chunk // 8, rbody, None)

            def body(i, carry):
                pl_ = i >> 5
                ch = i & 31
                bch = jnp.broadcast_to(ch, (LANES,))
                vals = [
                    plsc.load_gather(
                        rpk.at[pl.ds(pl_ * bw + g * LANES, LANES), :],
                        [iota, bch])
                    for g in range(ng)
                ]
                for g in range(ng):
                    tr[bb][pl_, ch >> 3, ch & 7,
                           pl.ds(g * LANES, LANES)] = vals[g]
                return carry

            lax.fori_loop(0, pc * D, body, None)

        start_gather(0, 0)
        start_gather(1, 1)
        wait_gather(0, 0)
        transpose_add(0, 0)
        start_store(0, 0)
        start_gather(0, 2)
        wait_gather(1, 1)
        transpose_add(1, 1)
        start_store(1, 1)

        def pair(cc, carry):
            c = cc * 2
            for b in range(2):
                wait_store(b, c + b - 2)
                start_gather(1 - b, c + b + 1)
                wait_gather(b, c + b)
                transpose_add(b, c + b)
                start_store(b, c + b)
            return carry

        lax.fori_loop(1, nchunk // 2 - 1, pair, None)
        c = nchunk - 2
        wait_store(0, c - 2)
        start_gather(1, c + 1)
        wait_gather(0, c)
        transpose_add(0, c)
        start_store(0, c)
        wait_store(1, c - 1)
        wait_gather(1, c + 1)
        transpose_add(1, c + 1)
        start_store(1, c + 1)
        wait_store(0, c)
        wait_store(1, c + 1)

    return k


def kernel(x, token_table, pos_table):
    B, M = x.shape
    D = token_table.shape[1]
    k = _build(B, M, D)
    out5d = k(x, token_table, pos_table)
    return jnp.transpose(out5d, (2, 4, 0, 1, 3)).reshape(B, M, D)
